# baseline (speedup 1.0000x reference)
import jax
import jax.numpy as jnp
from jax import lax
from jax.experimental import pallas as pl
from jax.experimental.pallas import tpu as pltpu
from jax.experimental.pallas import tpu_sc as plsc

N = 10000
D = 256
HALF = 128
W_ROW = 144
E = 160000
CHUNK = 64
NTILE = 16
EP = 163840
NCHUNK = EP // CHUNK
CPT = NCHUNK // NTILE
NACC = 10000
ROWS_PT = NACC // NTILE


NB = 4
ITILE = 8


def _sc_body(tab, gidx, didx, s_out, acc_sh, gi2_v, di2_v, rows, gsems, ssems,
             isem, zblk_v):
    cid = lax.axis_index("c")
    sid = lax.axis_index("s")
    z16 = jnp.zeros((16,), jnp.float32)

    for i in range(2):
        for j in range(W_ROW // 16):
            zblk_v[i, pl.ds(j * 16, 16)] = z16

    base = sid * ROWS_PT

    @pl.loop(0, ROWS_PT // 2)
    def _zero_acc(r):
        pltpu.sync_copy(zblk_v, acc_sh.at[pl.ds(base + r * 2, 2)])

    pltpu.sync_copy(zblk_v.at[pl.ds(0, 1)],
                    acc_sh.at[pl.ds(base + ROWS_PT - 1, 1)])

    plsc.subcore_barrier()

    NT2 = CPT // ITILE
    gbase = cid * NCHUNK + sid * CPT
    dbase = sid * CPT
    pltpu.sync_copy(gidx.at[pl.ds(gbase, ITILE)], gi2_v.at[0])
    pltpu.sync_copy(didx.at[pl.ds(dbase, ITILE)], di2_v.at[0])
    for b in range(NB):
        pltpu.make_async_copy(tab.at[gi2_v.at[0, b]], rows[b],
                              gsems[b]).start()

    @pl.loop(0, NT2)
    def _itile(t):
        p = lax.rem(t, 2)
        q = 1 - p
        has_next = t + 1 < NT2

        @pl.when(has_next)
        def _prefetch():
            c1 = (t + 1) * ITILE
            pltpu.make_async_copy(gidx.at[pl.ds(gbase + c1, ITILE)],
                                  gi2_v.at[q], isem).start()
            pltpu.make_async_copy(didx.at[pl.ds(dbase + c1, ITILE)],
                                  di2_v.at[q], isem).start()

        for j in range(ITILE // NB):
            k0 = j * NB
            for b in range(NB):
                pltpu.make_async_copy(tab.at[gi2_v.at[p, k0 + b]], rows[b],
                                      gsems[b]).wait()
                pltpu.async_copy(rows[b], acc_sh.at[di2_v.at[p, k0 + b]],
                                 ssems[b], add=True)
            for b in range(NB):
                kn = k0 + NB + b
                if kn < ITILE:
                    @pl.when((t < NT2 - 1) | (kn < ITILE))
                    def _refill(b=b, kn=kn):
                        pltpu.make_async_copy(rows[b],
                                              acc_sh.at[di2_v.at[p, kn]],
                                              ssems[b]).wait()
                        pltpu.make_async_copy(tab.at[gi2_v.at[p, kn]],
                                              rows[b], gsems[b]).start()
                else:
                    @pl.when(has_next)
                    def _refill_next(b=b, kn=kn):
                        if b == 0:
                            pltpu.make_async_copy(
                                gidx.at[pl.ds(gbase, ITILE)], gi2_v.at[q],
                                isem).wait()
                            pltpu.make_async_copy(
                                didx.at[pl.ds(dbase, ITILE)], di2_v.at[q],
                                isem).wait()
                        pltpu.make_async_copy(rows[b],
                                              acc_sh.at[di2_v.at[p, 0]],
                                              ssems[b]).wait()
                        pltpu.make_async_copy(tab.at[gi2_v.at[q, kn - ITILE]],
                                              rows[b], gsems[b]).start()

    for b in range(NB):
        pltpu.make_async_copy(rows[b], acc_sh.at[di2_v.at[0, b]],
                              ssems[b]).wait()

    plsc.subcore_barrier()

    pltpu.sync_copy(acc_sh.at[pl.ds(base, ROWS_PT)],
                    s_out.at[pl.ds(cid * NACC + base, ROWS_PT)])


_sc_kernel = pl.kernel(
    _sc_body,
    out_type=[jax.ShapeDtypeStruct((2 * NACC, W_ROW), jnp.float32)],
    mesh=plsc.VectorSubcoreMesh(core_axis_name="c", subcore_axis_name="s"),
    scratch_types=[
        pltpu.VMEM_SHARED((NACC, W_ROW), jnp.float32),
        pltpu.VMEM((2, ITILE, CHUNK), jnp.int32),
        pltpu.VMEM((2, ITILE, CHUNK), jnp.int32),
        [pltpu.VMEM((CHUNK, W_ROW), jnp.float32)] * NB,
        [pltpu.SemaphoreType.DMA] * NB,
        [pltpu.SemaphoreType.DMA] * NB,
        pltpu.SemaphoreType.DMA,
        pltpu.VMEM((2, W_ROW), jnp.float32),
    ],
    compiler_params=pltpu.CompilerParams(use_tc_tiling_on_sc=False),
)


RB = 400


def _tc_body(x_ref, s_ref, ws_ref, wm_ref, b_ref, o_ref):
    hi = lax.Precision.HIGHEST
    s0 = s_ref[0]
    s1 = s_ref[1]
    deg = jnp.maximum(s0[:, HALF:HALF + 1], 1.0)
    inv = 1.0 / deg
    acc = jnp.dot(x_ref[...], ws_ref[...], precision=hi,
                  preferred_element_type=jnp.float32)
    acc = acc + jnp.dot(s0[:, :HALF] * inv, wm_ref[0], precision=hi,
                        preferred_element_type=jnp.float32)
    acc = acc + jnp.dot(s1[:, :HALF] * inv, wm_ref[1], precision=hi,
                        preferred_element_type=jnp.float32)
    o_ref[...] = jnp.maximum(acc + b_ref[...], 0.0)


_tc_call = pl.pallas_call(
    _tc_body,
    grid=(N // RB,),
    in_specs=[
        pl.BlockSpec((RB, D), lambda i: (i, 0)),
        pl.BlockSpec((2, RB, W_ROW), lambda i: (0, i, 0)),
        pl.BlockSpec((D, D), lambda i: (0, 0)),
        pl.BlockSpec((2, HALF, D), lambda i: (0, 0, 0)),
        pl.BlockSpec((1, D), lambda i: (0, 0)),
    ],
    out_specs=pl.BlockSpec((RB, D), lambda i: (i, 0)),
    out_shape=jax.ShapeDtypeStruct((N, D), jnp.float32),
)


@jax.jit
def kernel(x_capec, edge_index_capec_to_capec, edge_index_capec_rel_capec,
           edge_label, W_self, W_msg, b, rel_embedding):
    del edge_index_capec_to_capec, edge_label, rel_embedding
    src = edge_index_capec_rel_capec[0].astype(jnp.int32)
    dst = edge_index_capec_rel_capec[1].astype(jnp.int32)
    pad = EP - E
    zrow = jnp.full((pad,), 2 * N, jnp.int32)
    g = src * 2
    gidx = jnp.concatenate([g, zrow, g + 1, zrow]).reshape(2 * NCHUNK, CHUNK)
    didx = jnp.concatenate(
        [dst, jnp.zeros((pad,), jnp.int32)]).reshape(NCHUNK, CHUNK)
    x2 = x_capec.reshape(2 * N, HALF)
    tab = jnp.concatenate(
        [x2, jnp.ones((2 * N, W_ROW - HALF), jnp.float32)], axis=1)
    tab = jnp.concatenate([tab, jnp.zeros((8, W_ROW), jnp.float32)], axis=0)
    (s_flat,) = _sc_kernel(tab, gidx, didx)
    s = s_flat.reshape(2, NACC, W_ROW)
    return _tc_call(x_capec, s, W_self,
                    W_msg.reshape(2, HALF, D), b.reshape(1, D))


# speedup vs baseline: 1.2273x; 1.2273x over previous
import jax
import jax.numpy as jnp
from jax import lax
from jax.experimental import pallas as pl
from jax.experimental.pallas import tpu as pltpu
from jax.experimental.pallas import tpu_sc as plsc

N = 10000
D = 256
HALF = 128
W_ROW = 144
E = 160000
CHUNK = 64
NTILE = 16
EP = 163840
NCHUNK = EP // CHUNK
CPT = NCHUNK // NTILE
NACC = 10048
ROWS_PT = NACC // NTILE


NB = 4
ITILE = 16


def _sc_body(tab, gidx, didx, s_out, acc_sh, gi2_v, di2_v, rows, gsems, ssems,
             zblk_v):
    cid = lax.axis_index("c")
    sid = lax.axis_index("s")
    z16 = jnp.zeros((16,), jnp.float32)

    for i in range(4):
        for j in range(W_ROW // 16):
            zblk_v[i, pl.ds(j * 16, 16)] = z16

    base = sid * ROWS_PT

    @pl.loop(0, ROWS_PT // 4)
    def _zero_acc(r):
        pltpu.sync_copy(zblk_v, acc_sh.at[pl.ds(base + r * 4, 4)])

    plsc.subcore_barrier()

    @pl.loop(0, CPT // ITILE)
    def _itile(t):
        c0 = t * ITILE
        pltpu.sync_copy(
            gidx.at[pl.ds(cid * NCHUNK + sid * CPT + c0, ITILE)], gi2_v)
        pltpu.sync_copy(didx.at[pl.ds(sid * CPT + c0, ITILE)], di2_v)

        for b in range(NB):
            pltpu.make_async_copy(tab.at[gi2_v.at[b]], rows[b],
                                  gsems[b]).start()

        @pl.loop(0, ITILE // NB)
        def _round(j):
            k0 = j * NB
            for b in range(NB):
                pltpu.make_async_copy(tab.at[gi2_v.at[k0 + b]], rows[b],
                                      gsems[b]).wait()
                pltpu.async_copy(rows[b], acc_sh.at[di2_v.at[k0 + b]],
                                 ssems[b], add=True)
            for b in range(NB):
                kn = k0 + NB + b

                @pl.when(kn < ITILE)
                def _refill(b=b, kn=kn):
                    pltpu.make_async_copy(rows[b], acc_sh.at[di2_v.at[kn]],
                                          ssems[b]).wait()
                    pltpu.make_async_copy(tab.at[gi2_v.at[kn]], rows[b],
                                          gsems[b]).start()

        for b in range(NB):
            pltpu.make_async_copy(rows[b], acc_sh.at[di2_v.at[b]],
                                  ssems[b]).wait()

    plsc.subcore_barrier()

    pltpu.sync_copy(acc_sh.at[pl.ds(base, ROWS_PT)],
                    s_out.at[pl.ds(cid * NACC + base, ROWS_PT)])


_sc_kernel = pl.kernel(
    _sc_body,
    out_type=[jax.ShapeDtypeStruct((2 * NACC, W_ROW), jnp.float32)],
    mesh=plsc.VectorSubcoreMesh(core_axis_name="c", subcore_axis_name="s"),
    scratch_types=[
        pltpu.VMEM_SHARED((NACC, W_ROW), jnp.float32),
        pltpu.VMEM((ITILE, CHUNK), jnp.int32),
        pltpu.VMEM((ITILE, CHUNK), jnp.int32),
        [pltpu.VMEM((CHUNK, W_ROW), jnp.float32)] * NB,
        [pltpu.SemaphoreType.DMA] * NB,
        [pltpu.SemaphoreType.DMA] * NB,
        pltpu.VMEM((4, W_ROW), jnp.float32),
    ],
    compiler_params=pltpu.CompilerParams(use_tc_tiling_on_sc=False),
)


RB = 400


def _tc_body(x_ref, s_ref, ws_ref, wm_ref, b_ref, o_ref):
    hi = lax.Precision.HIGHEST
    s0 = s_ref[0]
    s1 = s_ref[1]
    deg = jnp.maximum(s0[:, HALF:HALF + 1], 1.0)
    inv = 1.0 / deg
    acc = jnp.dot(x_ref[...], ws_ref[...], precision=hi,
                  preferred_element_type=jnp.float32)
    acc = acc + jnp.dot(s0[:, :HALF] * inv, wm_ref[0], precision=hi,
                        preferred_element_type=jnp.float32)
    acc = acc + jnp.dot(s1[:, :HALF] * inv, wm_ref[1], precision=hi,
                        preferred_element_type=jnp.float32)
    o_ref[...] = jnp.maximum(acc + b_ref[...], 0.0)


_tc_call = pl.pallas_call(
    _tc_body,
    grid=(N // RB,),
    in_specs=[
        pl.BlockSpec((RB, D), lambda i: (i, 0)),
        pl.BlockSpec((2, RB, W_ROW), lambda i: (0, i, 0)),
        pl.BlockSpec((D, D), lambda i: (0, 0)),
        pl.BlockSpec((2, HALF, D), lambda i: (0, 0, 0)),
        pl.BlockSpec((1, D), lambda i: (0, 0)),
    ],
    out_specs=pl.BlockSpec((RB, D), lambda i: (i, 0)),
    out_shape=jax.ShapeDtypeStruct((N, D), jnp.float32),
)


@jax.jit
def kernel(x_capec, edge_index_capec_to_capec, edge_index_capec_rel_capec,
           edge_label, W_self, W_msg, b, rel_embedding):
    del edge_index_capec_to_capec, edge_label, rel_embedding
    src = edge_index_capec_rel_capec[0].astype(jnp.int32)
    dst = edge_index_capec_rel_capec[1].astype(jnp.int32)
    pad = EP - E
    src_p = jnp.concatenate([src, jnp.zeros((pad,), jnp.int32)])
    dst_p = jnp.concatenate([dst, jnp.full((pad,), N, jnp.int32)])
    g = src_p * 2
    gidx = jnp.concatenate([g, g + 1]).reshape(2 * NCHUNK, CHUNK)
    didx = dst_p.reshape(NCHUNK, CHUNK)
    x2 = x_capec.reshape(2 * N, HALF)
    tab = jnp.concatenate(
        [x2, jnp.ones((2 * N, W_ROW - HALF), jnp.float32)], axis=1)
    (s_flat,) = _sc_kernel(tab, gidx, didx)
    s = s_flat.reshape(2, NACC, W_ROW)
    return _tc_call(x_capec, s, W_self,
                    W_msg.reshape(2, HALF, D), b.reshape(1, D))


# speedup vs baseline: 1.2453x; 1.0147x over previous
import jax
import jax.numpy as jnp
from jax import lax
from jax.experimental import pallas as pl
from jax.experimental.pallas import tpu as pltpu
from jax.experimental.pallas import tpu_sc as plsc

N = 10000
D = 256
HALF = 128
W_ROW = 144
E = 160000
CHUNK = 64
NTILE = 16
EP = 163840
NCHUNK = EP // CHUNK
CPT = NCHUNK // NTILE
NACC = 10048
ROWS_PT = NACC // NTILE


NB = 4
ITILE = 16


def _sc_body(tab, gidx, didx, s_out, acc_sh, gi2_v, di2_v, rows, gsems, ssems,
             zblk_v):
    cid = lax.axis_index("c")
    sid = lax.axis_index("s")
    z16 = jnp.zeros((16,), jnp.float32)

    for i in range(4):
        for j in range(W_ROW // 16):
            zblk_v[i, pl.ds(j * 16, 16)] = z16

    base = sid * ROWS_PT

    @pl.loop(0, ROWS_PT // 4)
    def _zero_acc(r):
        pltpu.sync_copy(zblk_v, acc_sh.at[pl.ds(base + r * 4, 4)])

    plsc.subcore_barrier()

    @pl.loop(0, CPT // ITILE)
    def _itile(t):
        c0 = t * ITILE
        pltpu.sync_copy(
            gidx.at[pl.ds(cid * NCHUNK + sid * CPT + c0, ITILE)], gi2_v)
        pltpu.sync_copy(didx.at[pl.ds(sid * CPT + c0, ITILE)], di2_v)

        for b in range(NB):
            pltpu.make_async_copy(tab.at[gi2_v.at[b]], rows[b],
                                  gsems[b]).start()

        @pl.loop(0, ITILE // NB)
        def _round(j):
            k0 = j * NB
            for b in range(NB):
                pltpu.make_async_copy(tab.at[gi2_v.at[k0 + b]], rows[b],
                                      gsems[b]).wait()
                pltpu.async_copy(rows[b], acc_sh.at[di2_v.at[k0 + b]],
                                 ssems[b], add=True)
            for b in range(NB):
                kn = k0 + NB + b

                @pl.when(kn < ITILE)
                def _refill(b=b, kn=kn):
                    pltpu.make_async_copy(rows[b], acc_sh.at[di2_v.at[kn]],
                                          ssems[b]).wait()
                    pltpu.make_async_copy(tab.at[gi2_v.at[kn]], rows[b],
                                          gsems[b]).start()

        for b in range(NB):
            pltpu.make_async_copy(rows[b], acc_sh.at[di2_v.at[b]],
                                  ssems[b]).wait()

    plsc.subcore_barrier()

    pltpu.sync_copy(acc_sh.at[pl.ds(base, ROWS_PT)],
                    s_out.at[pl.ds(cid * NACC + base, ROWS_PT)])


_sc_kernel = pl.kernel(
    _sc_body,
    out_type=[jax.ShapeDtypeStruct((2 * NACC, W_ROW), jnp.float32)],
    mesh=plsc.VectorSubcoreMesh(core_axis_name="c", subcore_axis_name="s"),
    scratch_types=[
        pltpu.VMEM_SHARED((NACC, W_ROW), jnp.float32),
        pltpu.VMEM((ITILE, CHUNK), jnp.int32),
        pltpu.VMEM((ITILE, CHUNK), jnp.int32),
        [pltpu.VMEM((CHUNK, W_ROW), jnp.float32)] * NB,
        [pltpu.SemaphoreType.DMA] * NB,
        [pltpu.SemaphoreType.DMA] * NB,
        pltpu.VMEM((4, W_ROW), jnp.float32),
    ],
    compiler_params=pltpu.CompilerParams(use_tc_tiling_on_sc=False),
)


RB = 1000


def _tc_body(x_ref, s_ref, ws_ref, wm_ref, b_ref, o_ref):
    hi = lax.Precision.HIGHEST
    s0 = s_ref[0]
    s1 = s_ref[1]
    deg = jnp.maximum(s0[:, HALF:HALF + 1], 1.0)
    inv = 1.0 / deg
    acc = jnp.dot(x_ref[...], ws_ref[...], precision=hi,
                  preferred_element_type=jnp.float32)
    acc = acc + jnp.dot(s0[:, :HALF] * inv, wm_ref[0], precision=hi,
                        preferred_element_type=jnp.float32)
    acc = acc + jnp.dot(s1[:, :HALF] * inv, wm_ref[1], precision=hi,
                        preferred_element_type=jnp.float32)
    o_ref[...] = jnp.maximum(acc + b_ref[...], 0.0)


_tc_call = pl.pallas_call(
    _tc_body,
    grid=(N // RB,),
    in_specs=[
        pl.BlockSpec((RB, D), lambda i: (i, 0)),
        pl.BlockSpec((2, RB, W_ROW), lambda i: (0, i, 0)),
        pl.BlockSpec((D, D), lambda i: (0, 0)),
        pl.BlockSpec((2, HALF, D), lambda i: (0, 0, 0)),
        pl.BlockSpec((1, D), lambda i: (0, 0)),
    ],
    out_specs=pl.BlockSpec((RB, D), lambda i: (i, 0)),
    out_shape=jax.ShapeDtypeStruct((N, D), jnp.float32),
)


@jax.jit
def kernel(x_capec, edge_index_capec_to_capec, edge_index_capec_rel_capec,
           edge_label, W_self, W_msg, b, rel_embedding):
    del edge_index_capec_to_capec, edge_label, rel_embedding
    src = edge_index_capec_rel_capec[0].astype(jnp.int32)
    dst = edge_index_capec_rel_capec[1].astype(jnp.int32)
    pad = EP - E
    src_p = jnp.concatenate([src, jnp.zeros((pad,), jnp.int32)])
    dst_p = jnp.concatenate([dst, jnp.full((pad,), N, jnp.int32)])
    g = src_p * 2
    gidx = jnp.concatenate([g, g + 1]).reshape(2 * NCHUNK, CHUNK)
    didx = dst_p.reshape(NCHUNK, CHUNK)
    x2 = x_capec.reshape(2 * N, HALF)
    tab = jnp.concatenate(
        [x2, jnp.ones((2 * N, W_ROW - HALF), jnp.float32)], axis=1)
    (s_flat,) = _sc_kernel(tab, gidx, didx)
    s = s_flat.reshape(2, NACC, W_ROW)
    return _tc_call(x_capec, s, W_self,
                    W_msg.reshape(2, HALF, D), b.reshape(1, D))
